# Initial kernel scaffold; baseline (speedup 1.0000x reference)
import jax
import jax.numpy as jnp
from jax.experimental import pallas as pl


def kernel(x, edge_index, W1, b1, W2, b2):
    raise NotImplementedError("write your pallas kernel here")


# speedup vs baseline: 31.6886x; 31.6886x over previous
import functools

import jax
import jax.numpy as jnp
from jax import lax
from jax.experimental import pallas as pl
from jax.experimental.pallas import tpu as pltpu
from jax.experimental.pallas import tpu_sc as plsc

NUM_CORES = 2
NUM_SUBCORES = 16
NUM_WORKERS = NUM_CORES * NUM_SUBCORES
LANES = 16
CHUNK = 128


def _vrsqrt(d):
    i = lax.bitcast_convert_type(d, jnp.int32)
    y = lax.bitcast_convert_type(jnp.int32(0x5F3759DF) - (i >> 1), jnp.float32)
    for _ in range(3):
        y = y * (1.5 - 0.5 * d * y * y)
    return y


def _make_deg_kernel(np_, n_chunks):
    rows_per_tile = np_ // NUM_SUBCORES
    vregs_per_tile = rows_per_tile // LANES
    chunks_per_tile = n_chunks // NUM_SUBCORES

    mesh = plsc.VectorSubcoreMesh(
        core_axis_name="c", subcore_axis_name="s",
        num_cores=NUM_CORES, num_subcores=NUM_SUBCORES)

    def body(dst_hbm, dis_hbm, deg_sh, idx_v, ones_v, deg_v, dis_v):
        c = lax.axis_index("c")
        s = lax.axis_index("s")

        @pl.when(c == 0)
        def _():
            def zb(i, carry):
                deg_v[pl.ds(i * LANES, LANES)] = jnp.zeros((LANES,), jnp.float32)
                return carry
            lax.fori_loop(0, vregs_per_tile, zb, None)
            pltpu.sync_copy(deg_v, deg_sh.at[pl.ds(s * rows_per_tile, rows_per_tile)])

            def ob(i, carry):
                ones_v[pl.ds(i * LANES, LANES)] = jnp.ones((LANES,), jnp.float32)
                return carry
            lax.fori_loop(0, CHUNK // LANES, ob, None)
            pltpu.sync_copy(dst_hbm.at[pl.ds(s * chunks_per_tile, chunks_per_tile)],
                            idx_v)

        plsc.subcore_barrier()

        @pl.when(c == 0)
        def _():
            def step(j, carry):
                pltpu.sync_copy(ones_v, deg_sh.at[idx_v.at[j]], add=True)
                return carry
            lax.fori_loop(0, chunks_per_tile, step, None)

        plsc.subcore_barrier()

        @pl.when(c == 0)
        def _():
            base = s * rows_per_tile
            pltpu.sync_copy(deg_sh.at[pl.ds(base, rows_per_tile)], deg_v)

            def dv(i, carry):
                d = deg_v[pl.ds(i * LANES, LANES)] + 1.0
                deg_v[pl.ds(i * LANES, LANES)] = _vrsqrt(d)
                return carry
            lax.fori_loop(0, vregs_per_tile, dv, None)

            def bc(i, carry):
                v = deg_v[pl.ds(i * LANES, LANES)]
                for l in range(LANES):
                    dis_v[i * LANES + l, :] = jnp.full((LANES,), v[l], jnp.float32)
                return carry
            lax.fori_loop(0, vregs_per_tile, bc, None)
            pltpu.sync_copy(dis_v, dis_hbm.at[pl.ds(base, rows_per_tile)])

    return pl.kernel(
        body,
        out_type=jax.ShapeDtypeStruct((np_, LANES), jnp.float32),
        mesh=mesh,
        compiler_params=pltpu.CompilerParams(use_tc_tiling_on_sc=False),
        scratch_types=[
            pltpu.VMEM_SHARED((np_,), jnp.float32),
            pltpu.VMEM((chunks_per_tile, CHUNK), jnp.int32),
            pltpu.VMEM((CHUNK,), jnp.float32),
            pltpu.VMEM((rows_per_tile,), jnp.float32),
            pltpu.VMEM((rows_per_tile, LANES), jnp.float32),
        ],
    )


def _make_agg_kernel(np_, n_chunks):
    rows_per_tile = np_ // NUM_SUBCORES
    cpw = n_chunks // NUM_WORKERS

    mesh = plsc.VectorSubcoreMesh(
        core_axis_name="c", subcore_axis_name="s",
        num_cores=NUM_CORES, num_subcores=NUM_SUBCORES)

    def body(g_hbm, src_hbm, dst_hbm, out_hbm, acc_sh, srcb, dstb, rows_v, zt, sem):
        c = lax.axis_index("c")
        s = lax.axis_index("s")
        w = s * NUM_CORES + c

        def zb(i, carry):
            zt[i, :] = jnp.zeros((LANES,), jnp.float32)
            return carry
        lax.fori_loop(0, rows_per_tile, zb, None)
        pltpu.sync_copy(zt, acc_sh.at[pl.ds(s * rows_per_tile, rows_per_tile)])
        pltpu.sync_copy(src_hbm.at[pl.ds(w * cpw, cpw)], srcb)
        pltpu.sync_copy(dst_hbm.at[pl.ds(w * cpw, cpw)], dstb)
        plsc.subcore_barrier()

        def step(j, carry):
            pltpu.async_copy(g_hbm.at[srcb.at[j]], rows_v, sem).wait()
            pltpu.sync_copy(rows_v, acc_sh.at[dstb.at[j]], add=True)
            return carry
        lax.fori_loop(0, cpw, step, None)

        plsc.subcore_barrier()
        base = s * rows_per_tile
        pltpu.sync_copy(acc_sh.at[pl.ds(base, rows_per_tile)],
                        out_hbm.at[pl.ds(c * np_ + base, rows_per_tile)])

    return pl.kernel(
        body,
        out_type=jax.ShapeDtypeStruct((NUM_CORES * np_, LANES), jnp.float32),
        mesh=mesh,
        compiler_params=pltpu.CompilerParams(use_tc_tiling_on_sc=False),
        scratch_types=[
            pltpu.VMEM_SHARED((np_, LANES), jnp.float32),
            pltpu.VMEM((cpw, CHUNK), jnp.int32),
            pltpu.VMEM((cpw, CHUNK), jnp.int32),
            pltpu.VMEM((CHUNK, LANES), jnp.float32),
            pltpu.VMEM((rows_per_tile, LANES), jnp.float32),
            pltpu.SemaphoreType.DMA,
        ],
    )


def _tc_scale_matmul(x, w1, dis2d, rb):
    n, f0 = x.shape
    f1 = w1.shape[1]
    grid = (n // rb,)

    def body(x_ref, w_ref, d_ref, o_ref):
        h = jnp.dot(x_ref[...], w_ref[...], preferred_element_type=jnp.float32)
        o_ref[...] = d_ref[...] * h

    return pl.pallas_call(
        body,
        grid=grid,
        in_specs=[
            pl.BlockSpec((rb, f0), lambda i: (i, 0)),
            pl.BlockSpec((f0, f1), lambda i: (0, 0)),
            pl.BlockSpec((rb, LANES), lambda i: (i, 0)),
        ],
        out_specs=pl.BlockSpec((rb, LANES), lambda i: (i, 0)),
        out_shape=jax.ShapeDtypeStruct((n, LANES), jnp.float32),
    )(x, w1, dis2d)


def _tc_mid_layer(p0, p1, g1, dis2d, b1, w2p, n, rb):
    grid = (n // rb,)

    def body(p0_ref, p1_ref, g_ref, d_ref, b_ref, w_ref, o_ref):
        agg = p0_ref[...] + p1_ref[...] + g_ref[...]
        h = jnp.maximum(d_ref[...] * agg + b_ref[...], 0.0)
        o_ref[...] = d_ref[...] * jnp.dot(
            h, w_ref[...], preferred_element_type=jnp.float32)

    blk = pl.BlockSpec((rb, LANES), lambda i: (i, 0))
    return pl.pallas_call(
        body,
        grid=grid,
        in_specs=[blk, blk, blk, blk,
                  pl.BlockSpec((1, LANES), lambda i: (0, 0)),
                  pl.BlockSpec((LANES, LANES), lambda i: (0, 0))],
        out_specs=blk,
        out_shape=jax.ShapeDtypeStruct((n, LANES), jnp.float32),
    )(p0, p1, g1, dis2d, b1, w2p)


def _tc_final_layer(q0, q1, g2, dis2d, b2p, n, rb):
    grid = (n // rb,)

    def body(q0_ref, q1_ref, g_ref, d_ref, b_ref, o_ref):
        agg = q0_ref[...] + q1_ref[...] + g_ref[...]
        o_ref[...] = jax.nn.sigmoid(d_ref[...] * agg + b_ref[...])

    blk = pl.BlockSpec((rb, LANES), lambda i: (i, 0))
    return pl.pallas_call(
        body,
        grid=grid,
        in_specs=[blk, blk, blk, blk,
                  pl.BlockSpec((1, LANES), lambda i: (0, 0))],
        out_specs=blk,
        out_shape=jax.ShapeDtypeStruct((n, LANES), jnp.float32),
    )(q0, q1, g2, dis2d, b2p)


def kernel(x, edge_index, W1, b1, W2, b2):
    n, f0 = x.shape
    e = edge_index.shape[1]
    f1 = W1.shape[1]
    f2 = W2.shape[1]

    np_ = -(-(n + LANES) // 256) * 256
    cpw = -(-(-(-e // (CHUNK * NUM_WORKERS))) // 8) * 8
    n_chunks = cpw * NUM_WORKERS
    e_pad = n_chunks * CHUNK

    src = edge_index[0]
    dst = edge_index[1]
    padi = jnp.arange(e_pad - e, dtype=src.dtype)
    src2d = jnp.concatenate([src, padi % LANES]).reshape(n_chunks, CHUNK)
    dst2d = jnp.concatenate([dst, n + (padi % LANES)]).reshape(n_chunks, CHUNK)

    w2p = jnp.zeros((f1, LANES), jnp.float32).at[:, :f2].set(W2)
    b1r = b1.reshape(1, f1)
    b2p = jnp.zeros((1, LANES), jnp.float32).at[0, :f2].set(b2)

    rb = 1000 if n % 1000 == 0 else 8

    dis2d = _make_deg_kernel(np_, n_chunks)(dst2d)
    agg = _make_agg_kernel(np_, n_chunks)

    g1 = _tc_scale_matmul(x, W1, dis2d, rb)
    p = agg(g1, src2d, dst2d)
    g2 = _tc_mid_layer(p[:np_], p[np_:], g1, dis2d, b1r, w2p, n, rb)
    q = agg(g2, src2d, dst2d)
    out = _tc_final_layer(q[:np_], q[np_:], g2, dis2d, b2p, n, rb)
    return out[:, :f2]


# speedup vs baseline: 42.0186x; 1.3260x over previous
import functools

import jax
import jax.numpy as jnp
from jax import lax
from jax.experimental import pallas as pl
from jax.experimental.pallas import tpu as pltpu
from jax.experimental.pallas import tpu_sc as plsc

NUM_CORES = 2
NUM_SUBCORES = 16
NUM_WORKERS = NUM_CORES * NUM_SUBCORES
LANES = 16
CHUNK = 128


def _make_deg_kernel(np_, n_chunks):
    rows_per_tile = np_ // NUM_SUBCORES
    vregs_per_tile = rows_per_tile // LANES
    cpw = n_chunks // NUM_WORKERS

    mesh = plsc.VectorSubcoreMesh(
        core_axis_name="c", subcore_axis_name="s",
        num_cores=NUM_CORES, num_subcores=NUM_SUBCORES)

    def body(dst_hbm, cnt_hbm, deg_sh, idx_v, ones_v, deg_v, cnt_v, sem):
        c = lax.axis_index("c")
        s = lax.axis_index("s")
        w = s * NUM_CORES + c

        def zb(i, carry):
            deg_v[pl.ds(i * LANES, LANES)] = jnp.zeros((LANES,), jnp.float32)
            return carry
        lax.fori_loop(0, vregs_per_tile, zb, None)
        pltpu.sync_copy(deg_v, deg_sh.at[pl.ds(s * rows_per_tile, rows_per_tile)])

        def ob(i, carry):
            ones_v[pl.ds(i * LANES, LANES)] = jnp.ones((LANES,), jnp.float32)
            return carry
        lax.fori_loop(0, CHUNK // LANES, ob, None)
        pltpu.sync_copy(dst_hbm.at[pl.ds(w * cpw, cpw)], idx_v)

        plsc.subcore_barrier()

        def fire(j, carry):
            pltpu.async_copy(ones_v, deg_sh.at[idx_v.at[j]], sem, add=True)
            return carry
        lax.fori_loop(0, cpw, fire, None)

        def drain(j, carry):
            pltpu.make_async_copy(ones_v, deg_sh.at[idx_v.at[0]], sem).wait()
            return carry
        lax.fori_loop(0, cpw, drain, None)

        plsc.subcore_barrier()

        base = s * rows_per_tile
        pltpu.sync_copy(deg_sh.at[pl.ds(base, rows_per_tile)], deg_v)

        def bc(i, carry):
            v = deg_v[pl.ds(i * LANES, LANES)]
            for l in range(LANES):
                cnt_v[i * LANES + l, :] = jnp.full((LANES,), v[l], jnp.float32)
            return carry
        lax.fori_loop(0, vregs_per_tile, bc, None)
        pltpu.sync_copy(cnt_v, cnt_hbm.at[pl.ds(c * np_ + base, rows_per_tile)])

    return pl.kernel(
        body,
        out_type=jax.ShapeDtypeStruct((NUM_CORES * np_, LANES), jnp.float32),
        mesh=mesh,
        compiler_params=pltpu.CompilerParams(use_tc_tiling_on_sc=False),
        scratch_types=[
            pltpu.VMEM_SHARED((np_,), jnp.float32),
            pltpu.VMEM((cpw, CHUNK), jnp.int32),
            pltpu.VMEM((CHUNK,), jnp.float32),
            pltpu.VMEM((rows_per_tile,), jnp.float32),
            pltpu.VMEM((rows_per_tile, LANES), jnp.float32),
            pltpu.SemaphoreType.DMA,
        ],
    )


def _make_agg_kernel(np_, n_chunks):
    rows_per_tile = np_ // NUM_SUBCORES
    cpw = n_chunks // NUM_WORKERS

    mesh = plsc.VectorSubcoreMesh(
        core_axis_name="c", subcore_axis_name="s",
        num_cores=NUM_CORES, num_subcores=NUM_SUBCORES)

    def body(g_hbm, src_hbm, dst_hbm, out_hbm, acc_sh, srcb, dstb,
             rows0, rows1, rows2, rows3, zt, gsem, ssem):
        c = lax.axis_index("c")
        s = lax.axis_index("s")
        w = s * NUM_CORES + c

        def zb(i, carry):
            zt[i, :] = jnp.zeros((LANES,), jnp.float32)
            return carry
        lax.fori_loop(0, rows_per_tile, zb, None)
        pltpu.sync_copy(zt, acc_sh.at[pl.ds(s * rows_per_tile, rows_per_tile)])
        pltpu.sync_copy(src_hbm.at[pl.ds(w * cpw, cpw)], srcb)
        pltpu.sync_copy(dst_hbm.at[pl.ds(w * cpw, cpw)], dstb)
        plsc.subcore_barrier()

        rows = (rows0, rows1, rows2, rows3)
        nbuf = len(rows)
        depth = nbuf - 1
        for k in range(depth):
            pltpu.async_copy(g_hbm.at[srcb.at[k]], rows[k], gsem)

        def group(jg, carry):
            for b in range(nbuf):
                j = jg * nbuf + b
                buf = rows[b]
                pltpu.make_async_copy(g_hbm.at[srcb.at[j]], buf, gsem).wait()
                pltpu.async_copy(buf, acc_sh.at[dstb.at[j]], ssem, add=True)
                prv = rows[(b - 1) % nbuf]

                @pl.when(j >= 1)
                def _():
                    pltpu.make_async_copy(prv, acc_sh.at[dstb.at[j - 1]],
                                          ssem).wait()

                @pl.when(j + depth < cpw)
                def _():
                    pltpu.async_copy(g_hbm.at[srcb.at[j + depth]], prv, gsem)
            return carry
        lax.fori_loop(0, cpw // nbuf, group, None)
        pltpu.make_async_copy(rows[(cpw - 1) % nbuf],
                              acc_sh.at[dstb.at[cpw - 1]], ssem).wait()

        plsc.subcore_barrier()
        base = s * rows_per_tile
        pltpu.sync_copy(acc_sh.at[pl.ds(base, rows_per_tile)],
                        out_hbm.at[pl.ds(c * np_ + base, rows_per_tile)])

    return pl.kernel(
        body,
        out_type=jax.ShapeDtypeStruct((NUM_CORES * np_, LANES), jnp.float32),
        mesh=mesh,
        compiler_params=pltpu.CompilerParams(use_tc_tiling_on_sc=False),
        scratch_types=[
            pltpu.VMEM_SHARED((np_, LANES), jnp.float32),
            pltpu.VMEM((cpw, CHUNK), jnp.int32),
            pltpu.VMEM((cpw, CHUNK), jnp.int32),
            pltpu.VMEM((CHUNK, LANES), jnp.float32),
            pltpu.VMEM((CHUNK, LANES), jnp.float32),
            pltpu.VMEM((CHUNK, LANES), jnp.float32),
            pltpu.VMEM((CHUNK, LANES), jnp.float32),
            pltpu.VMEM((rows_per_tile, LANES), jnp.float32),
            pltpu.SemaphoreType.DMA,
            pltpu.SemaphoreType.DMA,
        ],
    )


def _tc_scale_matmul(x, w1, c0, c1, rb):
    n, f0 = x.shape
    f1 = w1.shape[1]
    grid = (n // rb,)

    def body(x_ref, w_ref, c0_ref, c1_ref, o_ref, d_ref):
        dis = jax.lax.rsqrt(c0_ref[...] + c1_ref[...] + 1.0)
        d_ref[...] = dis
        h = jnp.dot(x_ref[...], w_ref[...], preferred_element_type=jnp.float32)
        o_ref[...] = dis * h

    blk = pl.BlockSpec((rb, LANES), lambda i: (i, 0))
    return pl.pallas_call(
        body,
        grid=grid,
        in_specs=[
            pl.BlockSpec((rb, f0), lambda i: (i, 0)),
            pl.BlockSpec((f0, f1), lambda i: (0, 0)),
            blk, blk,
        ],
        out_specs=[blk, blk],
        out_shape=[jax.ShapeDtypeStruct((n, LANES), jnp.float32),
                   jax.ShapeDtypeStruct((n, LANES), jnp.float32)],
    )(x, w1, c0, c1)


def _tc_mid_layer(p0, p1, g1, dis2d, b1, w2p, n, rb):
    grid = (n // rb,)

    def body(p0_ref, p1_ref, g_ref, d_ref, b_ref, w_ref, o_ref):
        agg = p0_ref[...] + p1_ref[...] + g_ref[...]
        h = jnp.maximum(d_ref[...] * agg + b_ref[...], 0.0)
        o_ref[...] = d_ref[...] * jnp.dot(
            h, w_ref[...], preferred_element_type=jnp.float32)

    blk = pl.BlockSpec((rb, LANES), lambda i: (i, 0))
    return pl.pallas_call(
        body,
        grid=grid,
        in_specs=[blk, blk, blk, blk,
                  pl.BlockSpec((1, LANES), lambda i: (0, 0)),
                  pl.BlockSpec((LANES, LANES), lambda i: (0, 0))],
        out_specs=blk,
        out_shape=jax.ShapeDtypeStruct((n, LANES), jnp.float32),
    )(p0, p1, g1, dis2d, b1, w2p)


def _tc_final_layer(q0, q1, g2, dis2d, b2p, n, rb):
    grid = (n // rb,)

    def body(q0_ref, q1_ref, g_ref, d_ref, b_ref, o_ref):
        agg = q0_ref[...] + q1_ref[...] + g_ref[...]
        o_ref[...] = jax.nn.sigmoid(d_ref[...] * agg + b_ref[...])

    blk = pl.BlockSpec((rb, LANES), lambda i: (i, 0))
    return pl.pallas_call(
        body,
        grid=grid,
        in_specs=[blk, blk, blk, blk,
                  pl.BlockSpec((1, LANES), lambda i: (0, 0))],
        out_specs=blk,
        out_shape=jax.ShapeDtypeStruct((n, LANES), jnp.float32),
    )(q0, q1, g2, dis2d, b2p)


def kernel(x, edge_index, W1, b1, W2, b2):
    n, f0 = x.shape
    e = edge_index.shape[1]
    f1 = W1.shape[1]
    f2 = W2.shape[1]

    np_ = -(-(n + LANES) // 256) * 256
    cpw = -(-(-(-e // (CHUNK * NUM_WORKERS))) // 8) * 8
    n_chunks = cpw * NUM_WORKERS
    e_pad = n_chunks * CHUNK

    src = edge_index[0]
    dst = edge_index[1]
    padi = jnp.arange(e_pad - e, dtype=src.dtype)
    src2d = jnp.concatenate([src, padi % LANES]).reshape(n_chunks, CHUNK)
    dst2d = jnp.concatenate([dst, n + (padi % LANES)]).reshape(n_chunks, CHUNK)

    w2p = jnp.zeros((f1, LANES), jnp.float32).at[:, :f2].set(W2)
    b1r = b1.reshape(1, f1)
    b2p = jnp.zeros((1, LANES), jnp.float32).at[0, :f2].set(b2)

    rb = 1000 if n % 1000 == 0 else 8

    cnt = _make_deg_kernel(np_, n_chunks)(dst2d)
    agg = _make_agg_kernel(np_, n_chunks)

    g1, dis2d = _tc_scale_matmul(x, W1, cnt[:np_], cnt[np_:], rb)
    p = agg(g1, src2d, dst2d)
    g2 = _tc_mid_layer(p[:np_], p[np_:], g1, dis2d, b1r, w2p, n, rb)
    q = agg(g2, src2d, dst2d)
    out = _tc_final_layer(q[:np_], q[np_:], g2, dis2d, b2p, n, rb)
    return out[:, :f2]


# speedup vs baseline: 45.6320x; 1.0860x over previous
import functools

import jax
import jax.numpy as jnp
from jax import lax
from jax.experimental import pallas as pl
from jax.experimental.pallas import tpu as pltpu
from jax.experimental.pallas import tpu_sc as plsc

NUM_CORES = 2
NUM_SUBCORES = 16
NUM_WORKERS = NUM_CORES * NUM_SUBCORES
LANES = 16
CHUNK = 128


def _make_deg_kernel(np_, n_chunks):
    rows_per_tile = np_ // NUM_SUBCORES
    vregs_per_tile = rows_per_tile // LANES
    cpw = n_chunks // NUM_WORKERS

    mesh = plsc.VectorSubcoreMesh(
        core_axis_name="c", subcore_axis_name="s",
        num_cores=NUM_CORES, num_subcores=NUM_SUBCORES)

    def body(dst_hbm, cnt_hbm, deg_sh, idx_v, ones_v, deg_v, cnt_v, sem):
        c = lax.axis_index("c")
        s = lax.axis_index("s")
        w = s * NUM_CORES + c

        def zb(i, carry):
            deg_v[pl.ds(i * LANES, LANES)] = jnp.zeros((LANES,), jnp.float32)
            return carry
        lax.fori_loop(0, vregs_per_tile, zb, None)
        pltpu.sync_copy(deg_v, deg_sh.at[pl.ds(s * rows_per_tile, rows_per_tile)])

        def ob(i, carry):
            ones_v[pl.ds(i * LANES, LANES)] = jnp.ones((LANES,), jnp.float32)
            return carry
        lax.fori_loop(0, CHUNK // LANES, ob, None)
        pltpu.sync_copy(dst_hbm.at[pl.ds(w * cpw, cpw)], idx_v)

        plsc.subcore_barrier()

        def fire(j, carry):
            pltpu.async_copy(ones_v, deg_sh.at[idx_v.at[j]], sem, add=True)
            return carry
        lax.fori_loop(0, cpw, fire, None)

        def drain(j, carry):
            pltpu.make_async_copy(ones_v, deg_sh.at[idx_v.at[0]], sem).wait()
            return carry
        lax.fori_loop(0, cpw, drain, None)

        plsc.subcore_barrier()

        base = s * rows_per_tile
        pltpu.sync_copy(deg_sh.at[pl.ds(base, rows_per_tile)], deg_v)

        def bc(i, carry):
            v = deg_v[pl.ds(i * LANES, LANES)]
            for l in range(LANES):
                cnt_v[i * LANES + l, :] = jnp.full((LANES,), v[l], jnp.float32)
            return carry
        lax.fori_loop(0, vregs_per_tile, bc, None)
        pltpu.sync_copy(cnt_v, cnt_hbm.at[c].at[pl.ds(base, rows_per_tile)])

    return pl.kernel(
        body,
        out_type=jax.ShapeDtypeStruct((NUM_CORES, np_, LANES), jnp.float32),
        mesh=mesh,
        compiler_params=pltpu.CompilerParams(use_tc_tiling_on_sc=False),
        scratch_types=[
            pltpu.VMEM_SHARED((np_,), jnp.float32),
            pltpu.VMEM((cpw, CHUNK), jnp.int32),
            pltpu.VMEM((CHUNK,), jnp.float32),
            pltpu.VMEM((rows_per_tile,), jnp.float32),
            pltpu.VMEM((rows_per_tile, LANES), jnp.float32),
            pltpu.SemaphoreType.DMA,
        ],
    )


def _make_agg_kernel(np_, n_chunks):
    rows_per_tile = np_ // NUM_SUBCORES
    cpw = n_chunks // NUM_WORKERS

    mesh = plsc.VectorSubcoreMesh(
        core_axis_name="c", subcore_axis_name="s",
        num_cores=NUM_CORES, num_subcores=NUM_SUBCORES)

    def body(g_hbm, src_hbm, dst_hbm, out_hbm, acc_sh, srcb, dstb,
             rows0, rows1, rows2, rows3, zt, gsem, ssem):
        c = lax.axis_index("c")
        s = lax.axis_index("s")
        w = s * NUM_CORES + c

        def zb(i, carry):
            zt[i, :] = jnp.zeros((LANES,), jnp.float32)
            return carry
        lax.fori_loop(0, rows_per_tile, zb, None)
        pltpu.sync_copy(zt, acc_sh.at[pl.ds(s * rows_per_tile, rows_per_tile)])
        pltpu.sync_copy(src_hbm.at[pl.ds(w * cpw, cpw)], srcb)
        pltpu.sync_copy(dst_hbm.at[pl.ds(w * cpw, cpw)], dstb)
        plsc.subcore_barrier()

        rows = (rows0, rows1, rows2, rows3)
        nbuf = len(rows)
        depth = nbuf - 1
        for k in range(depth):
            pltpu.async_copy(g_hbm.at[srcb.at[k]], rows[k], gsem)

        def group(jg, carry):
            for b in range(nbuf):
                j = jg * nbuf + b
                buf = rows[b]
                pltpu.make_async_copy(g_hbm.at[srcb.at[j]], buf, gsem).wait()
                pltpu.async_copy(buf, acc_sh.at[dstb.at[j]], ssem, add=True)
                prv = rows[(b - 1) % nbuf]

                @pl.when(j >= 1)
                def _():
                    pltpu.make_async_copy(prv, acc_sh.at[dstb.at[j - 1]],
                                          ssem).wait()

                @pl.when(j + depth < cpw)
                def _():
                    pltpu.async_copy(g_hbm.at[srcb.at[j + depth]], prv, gsem)
            return carry
        lax.fori_loop(0, cpw // nbuf, group, None)
        pltpu.make_async_copy(rows[(cpw - 1) % nbuf],
                              acc_sh.at[dstb.at[cpw - 1]], ssem).wait()

        plsc.subcore_barrier()
        base = s * rows_per_tile
        pltpu.sync_copy(acc_sh.at[pl.ds(base, rows_per_tile)],
                        out_hbm.at[c].at[pl.ds(base, rows_per_tile)])

    return pl.kernel(
        body,
        out_type=jax.ShapeDtypeStruct((NUM_CORES, np_, LANES), jnp.float32),
        mesh=mesh,
        compiler_params=pltpu.CompilerParams(use_tc_tiling_on_sc=False),
        scratch_types=[
            pltpu.VMEM_SHARED((np_, LANES), jnp.float32),
            pltpu.VMEM((cpw, CHUNK), jnp.int32),
            pltpu.VMEM((cpw, CHUNK), jnp.int32),
            pltpu.VMEM((CHUNK, LANES), jnp.float32),
            pltpu.VMEM((CHUNK, LANES), jnp.float32),
            pltpu.VMEM((CHUNK, LANES), jnp.float32),
            pltpu.VMEM((CHUNK, LANES), jnp.float32),
            pltpu.VMEM((rows_per_tile, LANES), jnp.float32),
            pltpu.SemaphoreType.DMA,
            pltpu.SemaphoreType.DMA,
        ],
    )


def _tc_scale_matmul(x, w1, cnt, rb):
    n, f0 = x.shape
    f1 = w1.shape[1]
    grid = (n // rb,)

    def body(x_ref, w_ref, c0_ref, c1_ref, o_ref, d_ref):
        dis = jax.lax.rsqrt(c0_ref[0] + c1_ref[0] + 1.0)
        d_ref[...] = dis
        h = jnp.dot(x_ref[...], w_ref[...], preferred_element_type=jnp.float32)
        o_ref[...] = dis * h

    blk = pl.BlockSpec((rb, LANES), lambda i: (i, 0))
    p0 = pl.BlockSpec((1, rb, LANES), lambda i: (0, i, 0))
    p1 = pl.BlockSpec((1, rb, LANES), lambda i: (1, i, 0))
    return pl.pallas_call(
        body,
        grid=grid,
        in_specs=[
            pl.BlockSpec((rb, f0), lambda i: (i, 0)),
            pl.BlockSpec((f0, f1), lambda i: (0, 0)),
            p0, p1,
        ],
        out_specs=[blk, blk],
        out_shape=[jax.ShapeDtypeStruct((n, LANES), jnp.float32),
                   jax.ShapeDtypeStruct((n, LANES), jnp.float32)],
    )(x, w1, cnt, cnt)


def _tc_mid_layer(p, g1, dis2d, b1, w2, n, rb):
    grid = (n // rb,)
    f1, f2 = w2.shape

    def body(p0_ref, p1_ref, g_ref, d_ref, b_ref, w_ref, o_ref):
        agg = p0_ref[0] + p1_ref[0] + g_ref[...]
        h = jnp.maximum(d_ref[...] * agg + b_ref[...], 0.0)
        o_ref[:, :f2] = d_ref[:, :f2] * jnp.dot(
            h, w_ref[...], preferred_element_type=jnp.float32)
        o_ref[:, f2:] = jnp.zeros((rb, LANES - f2), jnp.float32)

    blk = pl.BlockSpec((rb, LANES), lambda i: (i, 0))
    p0 = pl.BlockSpec((1, rb, LANES), lambda i: (0, i, 0))
    p1 = pl.BlockSpec((1, rb, LANES), lambda i: (1, i, 0))
    return pl.pallas_call(
        body,
        grid=grid,
        in_specs=[p0, p1, blk, blk,
                  pl.BlockSpec((1, LANES), lambda i: (0, 0)),
                  pl.BlockSpec((f1, f2), lambda i: (0, 0))],
        out_specs=blk,
        out_shape=jax.ShapeDtypeStruct((n, LANES), jnp.float32),
    )(p, p, g1, dis2d, b1.reshape(1, LANES), w2)


def _tc_final_layer(q, g2, dis2d, b2, n, rb, f2):
    grid = (n // rb,)

    def body(q0_ref, q1_ref, g_ref, d_ref, b_ref, o_ref):
        agg = q0_ref[0, :, :f2] + q1_ref[0, :, :f2] + g_ref[:, :f2]
        o_ref[...] = jax.nn.sigmoid(d_ref[:, :f2] * agg + b_ref[...])

    blk = pl.BlockSpec((rb, LANES), lambda i: (i, 0))
    p0 = pl.BlockSpec((1, rb, LANES), lambda i: (0, i, 0))
    p1 = pl.BlockSpec((1, rb, LANES), lambda i: (1, i, 0))
    return pl.pallas_call(
        body,
        grid=grid,
        in_specs=[p0, p1, blk, blk,
                  pl.BlockSpec((1, f2), lambda i: (0, 0))],
        out_specs=pl.BlockSpec((rb, f2), lambda i: (i, 0)),
        out_shape=jax.ShapeDtypeStruct((n, f2), jnp.float32),
    )(q, q, g2, dis2d, b2.reshape(1, f2))


def kernel(x, edge_index, W1, b1, W2, b2):
    n, f0 = x.shape
    e = edge_index.shape[1]
    f1 = W1.shape[1]
    f2 = W2.shape[1]

    np_ = -(-(n + LANES) // 256) * 256
    cpw = -(-(-(-e // (CHUNK * NUM_WORKERS))) // 8) * 8
    n_chunks = cpw * NUM_WORKERS
    e_pad = n_chunks * CHUNK

    src = edge_index[0]
    dst = edge_index[1]
    padi = jnp.arange(e_pad - e, dtype=src.dtype)
    src2d = jnp.concatenate([src, padi % LANES]).reshape(n_chunks, CHUNK)
    dst2d = jnp.concatenate([dst, n + (padi % LANES)]).reshape(n_chunks, CHUNK)

    rb = 1000 if n % 1000 == 0 else 8

    cnt = _make_deg_kernel(np_, n_chunks)(dst2d)
    agg = _make_agg_kernel(np_, n_chunks)

    g1, dis2d = _tc_scale_matmul(x, W1, cnt, rb)
    p = agg(g1, src2d, dst2d)
    g2 = _tc_mid_layer(p, g1, dis2d, b1, W2, n, rb)
    q = agg(g2, src2d, dst2d)
    return _tc_final_layer(q, g2, dis2d, b2, n, rb, f2)


# speedup vs baseline: 52.3068x; 1.1463x over previous
import functools

import jax
import jax.numpy as jnp
from jax import lax
from jax.experimental import pallas as pl
from jax.experimental.pallas import tpu as pltpu
from jax.experimental.pallas import tpu_sc as plsc

NUM_CORES = 2
NUM_SUBCORES = 16
NUM_WORKERS = NUM_CORES * NUM_SUBCORES
LANES = 16
CHUNK = 128


def _make_deg_kernel(np_, e, epw):
    rows_per_tile = np_ // NUM_SUBCORES
    vregs_per_tile = rows_per_tile // LANES
    fc = epw // CHUNK
    tail = epw - fc * CHUNK

    mesh = plsc.VectorSubcoreMesh(
        core_axis_name="c", subcore_axis_name="s",
        num_cores=NUM_CORES, num_subcores=NUM_SUBCORES)

    def body(ei_hbm, cnt_hbm, deg_sh, idx_v, ones_v, deg_v, cnt_v, tidx_v,
             isem, ssem):
        c = lax.axis_index("c")
        s = lax.axis_index("s")
        w = s * NUM_CORES + c
        ebase = w * epw

        def zb(i, carry):
            deg_v[pl.ds(i * LANES, LANES)] = jnp.zeros((LANES,), jnp.float32)
            return carry
        lax.fori_loop(0, vregs_per_tile, zb, None)
        pltpu.sync_copy(deg_v, deg_sh.at[pl.ds(s * rows_per_tile, rows_per_tile)])

        def ob(i, carry):
            ones_v[pl.ds(i * LANES, LANES)] = jnp.ones((LANES,), jnp.float32)
            return carry
        lax.fori_loop(0, CHUNK // LANES, ob, None)
        pltpu.sync_copy(ei_hbm.at[1].at[pl.ds(ebase, epw)], idx_v)

        plsc.subcore_barrier()

        def fire(j, carry):
            pltpu.async_copy(ones_v, deg_sh.at[idx_v.at[pl.ds(j * CHUNK, CHUNK)]],
                             ssem, add=True)
            return carry
        lax.fori_loop(0, fc, fire, None)

        def drain(j, carry):
            pltpu.make_async_copy(ones_v, deg_sh.at[idx_v.at[pl.ds(0, CHUNK)]],
                                  ssem).wait()
            return carry
        lax.fori_loop(0, fc, drain, None)

        if tail:
            pltpu.sync_copy(ei_hbm.at[1].at[pl.ds(ebase + fc * CHUNK, tail)],
                            tidx_v)
            pltpu.sync_copy(ones_v.at[pl.ds(0, tail)], deg_sh.at[tidx_v],
                            add=True)

        plsc.subcore_barrier()

        base = s * rows_per_tile
        pltpu.sync_copy(deg_sh.at[pl.ds(base, rows_per_tile)], deg_v)

        def bc(i, carry):
            v = deg_v[pl.ds(i * LANES, LANES)]
            for l in range(LANES):
                cnt_v[i * LANES + l, :] = jnp.full((LANES,), v[l], jnp.float32)
            return carry
        lax.fori_loop(0, vregs_per_tile, bc, None)
        pltpu.sync_copy(cnt_v, cnt_hbm.at[c].at[pl.ds(base, rows_per_tile)])

    return pl.kernel(
        body,
        out_type=jax.ShapeDtypeStruct((NUM_CORES, np_, LANES), jnp.float32),
        mesh=mesh,
        compiler_params=pltpu.CompilerParams(use_tc_tiling_on_sc=False),
        scratch_types=[
            pltpu.VMEM_SHARED((np_,), jnp.float32),
            pltpu.VMEM((epw,), jnp.int32),
            pltpu.VMEM((CHUNK,), jnp.float32),
            pltpu.VMEM((rows_per_tile,), jnp.float32),
            pltpu.VMEM((rows_per_tile, LANES), jnp.float32),
            pltpu.VMEM((tail if tail else LANES,), jnp.int32),
            pltpu.SemaphoreType.DMA,
            pltpu.SemaphoreType.DMA,
        ],
    )


def _make_agg_kernel(np_, e, epw):
    rows_per_tile = np_ // NUM_SUBCORES
    fc = epw // CHUNK
    tail = epw - fc * CHUNK

    mesh = plsc.VectorSubcoreMesh(
        core_axis_name="c", subcore_axis_name="s",
        num_cores=NUM_CORES, num_subcores=NUM_SUBCORES)

    def body(g_hbm, ei_hbm, out_hbm, acc_sh, srcb, dstb,
             rows0, rows1, rows2, rows3, zt, tidx_v, trows, gsem, ssem):
        c = lax.axis_index("c")
        s = lax.axis_index("s")
        w = s * NUM_CORES + c
        ebase = w * epw

        def zb(i, carry):
            zt[i, :] = jnp.zeros((LANES,), jnp.float32)
            return carry
        lax.fori_loop(0, rows_per_tile, zb, None)
        pltpu.sync_copy(zt, acc_sh.at[pl.ds(s * rows_per_tile, rows_per_tile)])
        pltpu.sync_copy(ei_hbm.at[0].at[pl.ds(ebase, epw)], srcb)
        pltpu.sync_copy(ei_hbm.at[1].at[pl.ds(ebase, epw)], dstb)
        plsc.subcore_barrier()

        rows = (rows0, rows1, rows2, rows3)
        nbuf = len(rows)
        depth = nbuf - 1

        def sidx(j):
            return srcb.at[pl.ds(j * CHUNK, CHUNK)]

        def didx(j):
            return dstb.at[pl.ds(j * CHUNK, CHUNK)]

        for k in range(depth):
            pltpu.async_copy(g_hbm.at[sidx(k)], rows[k], gsem)

        def group(jg, carry):
            for b in range(nbuf):
                j = jg * nbuf + b
                buf = rows[b]
                pltpu.make_async_copy(g_hbm.at[sidx(j)], buf, gsem).wait()
                pltpu.async_copy(buf, acc_sh.at[didx(j)], ssem, add=True)
                prv = rows[(b - 1) % nbuf]

                @pl.when(j >= 1)
                def _():
                    pltpu.make_async_copy(prv, acc_sh.at[didx(j - 1)],
                                          ssem).wait()

                @pl.when(j + depth < fc)
                def _():
                    pltpu.async_copy(g_hbm.at[sidx(j + depth)], prv, gsem)
            return carry
        lax.fori_loop(0, fc // nbuf, group, None)
        for j in range(fc - fc % nbuf, fc):
            b = j % nbuf
            pltpu.make_async_copy(g_hbm.at[sidx(j)], rows[b], gsem).wait()
            pltpu.async_copy(rows[b], acc_sh.at[didx(j)], ssem, add=True)
            pltpu.make_async_copy(rows[(b - 1) % nbuf],
                                  acc_sh.at[didx(j - 1)], ssem).wait()
        pltpu.make_async_copy(rows[(fc - 1) % nbuf],
                              acc_sh.at[didx(fc - 1)], ssem).wait()

        if tail:
            pltpu.sync_copy(ei_hbm.at[0].at[pl.ds(ebase + fc * CHUNK, tail)],
                            tidx_v)
            pltpu.async_copy(g_hbm.at[tidx_v], trows, gsem).wait()
            pltpu.sync_copy(ei_hbm.at[1].at[pl.ds(ebase + fc * CHUNK, tail)],
                            tidx_v)
            pltpu.sync_copy(trows, acc_sh.at[tidx_v], add=True)

        plsc.subcore_barrier()
        base = s * rows_per_tile
        pltpu.sync_copy(acc_sh.at[pl.ds(base, rows_per_tile)],
                        out_hbm.at[c].at[pl.ds(base, rows_per_tile)])

    return pl.kernel(
        body,
        out_type=jax.ShapeDtypeStruct((NUM_CORES, np_, LANES), jnp.float32),
        mesh=mesh,
        compiler_params=pltpu.CompilerParams(use_tc_tiling_on_sc=False),
        scratch_types=[
            pltpu.VMEM_SHARED((np_, LANES), jnp.float32),
            pltpu.VMEM((epw,), jnp.int32),
            pltpu.VMEM((epw,), jnp.int32),
            pltpu.VMEM((CHUNK, LANES), jnp.float32),
            pltpu.VMEM((CHUNK, LANES), jnp.float32),
            pltpu.VMEM((CHUNK, LANES), jnp.float32),
            pltpu.VMEM((CHUNK, LANES), jnp.float32),
            pltpu.VMEM((rows_per_tile, LANES), jnp.float32),
            pltpu.VMEM((tail if tail else LANES,), jnp.int32),
            pltpu.VMEM((tail if tail else LANES, LANES), jnp.float32),
            pltpu.SemaphoreType.DMA,
            pltpu.SemaphoreType.DMA,
        ],
    )


def _tc_scale_matmul(x, w1, cnt, rb):
    n, f0 = x.shape
    f1 = w1.shape[1]
    grid = (n // rb,)

    def body(x_ref, w_ref, c0_ref, c1_ref, o_ref, d_ref):
        dis = jax.lax.rsqrt(c0_ref[0] + c1_ref[0] + 1.0)
        d_ref[...] = dis
        h = jnp.dot(x_ref[...], w_ref[...], preferred_element_type=jnp.float32)
        o_ref[...] = dis * h

    blk = pl.BlockSpec((rb, LANES), lambda i: (i, 0))
    p0 = pl.BlockSpec((1, rb, LANES), lambda i: (0, i, 0))
    p1 = pl.BlockSpec((1, rb, LANES), lambda i: (1, i, 0))
    return pl.pallas_call(
        body,
        grid=grid,
        in_specs=[
            pl.BlockSpec((rb, f0), lambda i: (i, 0)),
            pl.BlockSpec((f0, f1), lambda i: (0, 0)),
            p0, p1,
        ],
        out_specs=[blk, blk],
        out_shape=[jax.ShapeDtypeStruct((n, LANES), jnp.float32),
                   jax.ShapeDtypeStruct((n, LANES), jnp.float32)],
    )(x, w1, cnt, cnt)


def _tc_mid_layer(p, g1, dis2d, b1, w2, n, rb):
    grid = (n // rb,)
    f1, f2 = w2.shape

    def body(p0_ref, p1_ref, g_ref, d_ref, b_ref, w_ref, o_ref):
        agg = p0_ref[0] + p1_ref[0] + g_ref[...]
        h = jnp.maximum(d_ref[...] * agg + b_ref[...], 0.0)
        o_ref[:, :f2] = d_ref[:, :f2] * jnp.dot(
            h, w_ref[...], preferred_element_type=jnp.float32)
        o_ref[:, f2:] = jnp.zeros((rb, LANES - f2), jnp.float32)

    blk = pl.BlockSpec((rb, LANES), lambda i: (i, 0))
    p0 = pl.BlockSpec((1, rb, LANES), lambda i: (0, i, 0))
    p1 = pl.BlockSpec((1, rb, LANES), lambda i: (1, i, 0))
    return pl.pallas_call(
        body,
        grid=grid,
        in_specs=[p0, p1, blk, blk,
                  pl.BlockSpec((1, LANES), lambda i: (0, 0)),
                  pl.BlockSpec((f1, f2), lambda i: (0, 0))],
        out_specs=blk,
        out_shape=jax.ShapeDtypeStruct((n, LANES), jnp.float32),
    )(p, p, g1, dis2d, b1.reshape(1, LANES), w2)


def _tc_final_layer(q, g2, dis2d, b2, n, rb, f2):
    grid = (n // rb,)

    def body(q0_ref, q1_ref, g_ref, d_ref, b_ref, o_ref):
        agg = q0_ref[0, :, :f2] + q1_ref[0, :, :f2] + g_ref[:, :f2]
        o_ref[...] = jax.nn.sigmoid(d_ref[:, :f2] * agg + b_ref[...])

    blk = pl.BlockSpec((rb, LANES), lambda i: (i, 0))
    p0 = pl.BlockSpec((1, rb, LANES), lambda i: (0, i, 0))
    p1 = pl.BlockSpec((1, rb, LANES), lambda i: (1, i, 0))
    return pl.pallas_call(
        body,
        grid=grid,
        in_specs=[p0, p1, blk, blk,
                  pl.BlockSpec((1, f2), lambda i: (0, 0))],
        out_specs=pl.BlockSpec((rb, f2), lambda i: (i, 0)),
        out_shape=jax.ShapeDtypeStruct((n, f2), jnp.float32),
    )(q, q, g2, dis2d, b2.reshape(1, f2))


def kernel(x, edge_index, W1, b1, W2, b2):
    n, f0 = x.shape
    e = edge_index.shape[1]
    f1 = W1.shape[1]
    f2 = W2.shape[1]

    np_ = -(-(n + LANES) // 256) * 256
    if e % (NUM_WORKERS * LANES):
        e_pad = -(-e // (NUM_WORKERS * LANES)) * NUM_WORKERS * LANES
        padi = jnp.arange(e_pad - e, dtype=edge_index.dtype)
        ei = jnp.concatenate(
            [edge_index, jnp.stack([padi % LANES, n + (padi % LANES)])], axis=1)
    else:
        e_pad = e
        ei = edge_index
    epw = e_pad // NUM_WORKERS

    rb = 1000 if n % 1000 == 0 else 8

    cnt = _make_deg_kernel(np_, e_pad, epw)(ei)
    agg = _make_agg_kernel(np_, e_pad, epw)

    g1, dis2d = _tc_scale_matmul(x, W1, cnt, rb)
    p = agg(g1, ei)
    g2 = _tc_mid_layer(p, g1, dis2d, b1, W2, n, rb)
    q = agg(g2, ei)
    return _tc_final_layer(q, g2, dis2d, b2, n, rb, f2)


# speedup vs baseline: 74.5835x; 1.4259x over previous
import functools

import jax
import jax.numpy as jnp
from jax import lax
from jax.experimental import pallas as pl
from jax.experimental.pallas import tpu as pltpu
from jax.experimental.pallas import tpu_sc as plsc

NUM_CORES = 2
NUM_SUBCORES = 16
NUM_WORKERS = NUM_CORES * NUM_SUBCORES
LANES = 16
CHUNK = 128


def _make_deg_kernel(np_, e, epw):
    rows_per_tile = np_ // NUM_SUBCORES
    vregs_per_tile = rows_per_tile // LANES
    fc = epw // CHUNK
    tail = epw - fc * CHUNK

    mesh = plsc.VectorSubcoreMesh(
        core_axis_name="c", subcore_axis_name="s",
        num_cores=NUM_CORES, num_subcores=NUM_SUBCORES)

    def body(ei_hbm, cnt_hbm, deg_sh, idx_v, ones_v, deg_v, cnt_v, tidx_v,
             isem, ssem):
        c = lax.axis_index("c")
        s = lax.axis_index("s")
        w = s * NUM_CORES + c
        ebase = w * epw

        def zb(i, carry):
            deg_v[pl.ds(i * LANES, LANES)] = jnp.zeros((LANES,), jnp.float32)
            return carry
        lax.fori_loop(0, vregs_per_tile, zb, None)
        pltpu.sync_copy(deg_v, deg_sh.at[pl.ds(s * rows_per_tile, rows_per_tile)])

        def ob(i, carry):
            ones_v[pl.ds(i * LANES, LANES)] = jnp.ones((LANES,), jnp.float32)
            return carry
        lax.fori_loop(0, CHUNK // LANES, ob, None)
        pltpu.sync_copy(ei_hbm.at[1].at[pl.ds(ebase, epw)], idx_v)

        plsc.subcore_barrier()

        def fire(j, carry):
            pltpu.async_copy(ones_v, deg_sh.at[idx_v.at[pl.ds(j * CHUNK, CHUNK)]],
                             ssem, add=True)
            return carry
        lax.fori_loop(0, fc, fire, None)

        def drain(j, carry):
            pltpu.make_async_copy(ones_v, deg_sh.at[idx_v.at[pl.ds(0, CHUNK)]],
                                  ssem).wait()
            return carry
        lax.fori_loop(0, fc, drain, None)

        if tail:
            pltpu.sync_copy(ei_hbm.at[1].at[pl.ds(ebase + fc * CHUNK, tail)],
                            tidx_v)
            pltpu.sync_copy(ones_v.at[pl.ds(0, tail)], deg_sh.at[tidx_v],
                            add=True)

        plsc.subcore_barrier()

        base = s * rows_per_tile
        pltpu.sync_copy(deg_sh.at[pl.ds(base, rows_per_tile)], deg_v)

        def bc(i, carry):
            v = deg_v[pl.ds(i * LANES, LANES)]
            for l in range(LANES):
                cnt_v[i * LANES + l, :] = jnp.full((LANES,), v[l], jnp.float32)
            return carry
        lax.fori_loop(0, vregs_per_tile, bc, None)
        pltpu.sync_copy(cnt_v, cnt_hbm.at[c].at[pl.ds(base, rows_per_tile)])

    return pl.kernel(
        body,
        out_type=jax.ShapeDtypeStruct((NUM_CORES, np_, LANES), jnp.float32),
        mesh=mesh,
        compiler_params=pltpu.CompilerParams(use_tc_tiling_on_sc=False),
        scratch_types=[
            pltpu.VMEM_SHARED((np_,), jnp.float32),
            pltpu.VMEM((epw,), jnp.int32),
            pltpu.VMEM((CHUNK,), jnp.float32),
            pltpu.VMEM((rows_per_tile,), jnp.float32),
            pltpu.VMEM((rows_per_tile, LANES), jnp.float32),
            pltpu.VMEM((tail if tail else LANES,), jnp.int32),
            pltpu.SemaphoreType.DMA,
            pltpu.SemaphoreType.DMA,
        ],
    )


def _make_agg_kernel(np_, e, epw):
    rows_per_tile = np_ // NUM_SUBCORES
    fc = epw // CHUNK
    tail = epw - fc * CHUNK

    mesh = plsc.VectorSubcoreMesh(
        core_axis_name="c", subcore_axis_name="s",
        num_cores=NUM_CORES, num_subcores=NUM_SUBCORES)

    def body(g_hbm, ei_hbm, out_hbm, acc_sh, srcb, dstb,
             rows0, rows1, rows2, rows3, zt, tidx_v, trows, gsem, ssem):
        c = lax.axis_index("c")
        s = lax.axis_index("s")
        w = s * NUM_CORES + c
        ebase = w * epw

        def zb(i, carry):
            zt[i, :] = jnp.zeros((LANES,), jnp.float32)
            return carry
        lax.fori_loop(0, rows_per_tile, zb, None)
        pltpu.sync_copy(zt, acc_sh.at[pl.ds(s * rows_per_tile, rows_per_tile)])
        pltpu.sync_copy(ei_hbm.at[0].at[pl.ds(ebase, epw)], srcb)
        pltpu.sync_copy(ei_hbm.at[1].at[pl.ds(ebase, epw)], dstb)
        plsc.subcore_barrier()

        rows = (rows0, rows1, rows2, rows3)
        nbuf = len(rows)
        depth = nbuf - 1

        def sidx(j):
            return srcb.at[pl.ds(j * CHUNK, CHUNK)]

        def didx(j):
            return dstb.at[pl.ds(j * CHUNK, CHUNK)]

        for k in range(depth):
            pltpu.async_copy(g_hbm.at[sidx(k)], rows[k], gsem)

        def group(jg, carry):
            for b in range(nbuf):
                j = jg * nbuf + b
                buf = rows[b]
                pltpu.make_async_copy(g_hbm.at[sidx(j)], buf, gsem).wait()
                pltpu.async_copy(buf, acc_sh.at[didx(j)], ssem, add=True)
                prv = rows[(b - 1) % nbuf]

                @pl.when(j >= 1)
                def _():
                    pltpu.make_async_copy(prv, acc_sh.at[didx(j - 1)],
                                          ssem).wait()

                @pl.when(j + depth < fc)
                def _():
                    pltpu.async_copy(g_hbm.at[sidx(j + depth)], prv, gsem)
            return carry
        lax.fori_loop(0, fc // nbuf, group, None)
        for j in range(fc - fc % nbuf, fc):
            b = j % nbuf
            pltpu.make_async_copy(g_hbm.at[sidx(j)], rows[b], gsem).wait()
            pltpu.async_copy(rows[b], acc_sh.at[didx(j)], ssem, add=True)
            pltpu.make_async_copy(rows[(b - 1) % nbuf],
                                  acc_sh.at[didx(j - 1)], ssem).wait()
        pltpu.make_async_copy(rows[(fc - 1) % nbuf],
                              acc_sh.at[didx(fc - 1)], ssem).wait()

        if tail:
            pltpu.sync_copy(ei_hbm.at[0].at[pl.ds(ebase + fc * CHUNK, tail)],
                            tidx_v)
            pltpu.async_copy(g_hbm.at[tidx_v], trows, gsem).wait()
            pltpu.sync_copy(ei_hbm.at[1].at[pl.ds(ebase + fc * CHUNK, tail)],
                            tidx_v)
            pltpu.sync_copy(trows, acc_sh.at[tidx_v], add=True)

        plsc.subcore_barrier()
        base = s * rows_per_tile
        pltpu.sync_copy(acc_sh.at[pl.ds(base, rows_per_tile)],
                        out_hbm.at[c].at[pl.ds(base, rows_per_tile)])

    return pl.kernel(
        body,
        out_type=jax.ShapeDtypeStruct((NUM_CORES, np_, LANES), jnp.float32),
        mesh=mesh,
        compiler_params=pltpu.CompilerParams(use_tc_tiling_on_sc=False),
        scratch_types=[
            pltpu.VMEM_SHARED((np_, LANES), jnp.float32),
            pltpu.VMEM((epw,), jnp.int32),
            pltpu.VMEM((epw,), jnp.int32),
            pltpu.VMEM((CHUNK, LANES), jnp.float32),
            pltpu.VMEM((CHUNK, LANES), jnp.float32),
            pltpu.VMEM((CHUNK, LANES), jnp.float32),
            pltpu.VMEM((CHUNK, LANES), jnp.float32),
            pltpu.VMEM((rows_per_tile, LANES), jnp.float32),
            pltpu.VMEM((tail if tail else LANES,), jnp.int32),
            pltpu.VMEM((tail if tail else LANES, LANES), jnp.float32),
            pltpu.SemaphoreType.DMA,
            pltpu.SemaphoreType.DMA,
        ],
    )


_FW = 128
_GRP = _FW // LANES


def _tc_scale_matmul(x8, w1b, cnt_f, nf, npf):
    def body(x_ref, w_ref, c_ref, o_ref, d_ref):
        dis = jax.lax.rsqrt(c_ref[0, :nf] + c_ref[1, :nf] + 1.0)
        d_ref[...] = dis
        h = jnp.dot(x_ref[...], w_ref[...], preferred_element_type=jnp.float32)
        o_ref[...] = dis * h

    return pl.pallas_call(
        body,
        out_shape=[jax.ShapeDtypeStruct((nf, _FW), jnp.float32),
                   jax.ShapeDtypeStruct((nf, _FW), jnp.float32)],
    )(x8, w1b, cnt_f)


def _tc_mid_layer(p_f, g1_f, dis_f, b1, w2b, nf):
    def body(p_ref, g_ref, d_ref, b_ref, w_ref, o_ref):
        agg = p_ref[0, :nf] + p_ref[1, :nf] + g_ref[...]
        h = jnp.maximum(d_ref[...] * agg + b_ref[...], 0.0)
        o_ref[...] = d_ref[...] * jnp.dot(
            h, w_ref[...], preferred_element_type=jnp.float32)

    return pl.pallas_call(
        body,
        out_shape=jax.ShapeDtypeStruct((nf, _FW), jnp.float32),
    )(p_f, g1_f, dis_f, b1, w2b)


def _tc_final_layer(q_f, g2_f, dis_f, b2, nf):
    def body(q_ref, g_ref, d_ref, b_ref, o_ref):
        agg = q_ref[0, :nf] + q_ref[1, :nf] + g_ref[...]
        o_ref[...] = jax.nn.sigmoid(d_ref[...] * agg + b_ref[...])

    return pl.pallas_call(
        body,
        out_shape=jax.ShapeDtypeStruct((nf, _FW), jnp.float32),
    )(q_f, g2_f, dis_f, b2)


def kernel(x, edge_index, W1, b1, W2, b2):
    n, f0 = x.shape
    e = edge_index.shape[1]
    f1 = W1.shape[1]
    f2 = W2.shape[1]

    np_ = -(-(n + LANES) // 256) * 256
    if e % (NUM_WORKERS * LANES):
        e_pad = -(-e // (NUM_WORKERS * LANES)) * NUM_WORKERS * LANES
        padi = jnp.arange(e_pad - e, dtype=edge_index.dtype)
        ei = jnp.concatenate(
            [edge_index, jnp.stack([padi % LANES, n + (padi % LANES)])], axis=1)
    else:
        e_pad = e
        ei = edge_index
    epw = e_pad // NUM_WORKERS
    nf = n * LANES // _FW
    npf = np_ * LANES // _FW

    b1row = jnp.tile(b1, _GRP).reshape(1, _FW)
    w1b = jnp.kron(jnp.eye(_GRP, dtype=jnp.float32), W1)
    w2pad = jnp.zeros((f1, LANES), jnp.float32).at[:, :f2].set(W2)
    w2b = jnp.kron(jnp.eye(_GRP, dtype=jnp.float32), w2pad)
    b2row = jnp.tile(jnp.zeros((LANES,), jnp.float32).at[:f2].set(b2),
                     _GRP).reshape(1, _FW)

    cnt = _make_deg_kernel(np_, e_pad, epw)(ei)
    agg = _make_agg_kernel(np_, e_pad, epw)

    g1_f, dis_f = _tc_scale_matmul(x.reshape(nf, f0 * _GRP), w1b,
                                   cnt.reshape(2, npf, _FW), nf, npf)
    p = agg(g1_f.reshape(n, LANES), ei)
    g2_f = _tc_mid_layer(p.reshape(2, npf, _FW), g1_f, dis_f, b1row, w2b, nf)
    q = agg(g2_f.reshape(n, LANES), ei)
    out_f = _tc_final_layer(q.reshape(2, npf, _FW), g2_f, dis_f, b2row, nf)
    return out_f.reshape(n, LANES)[:, :f2]


# speedup vs baseline: 100.5545x; 1.3482x over previous
import functools

import jax
import jax.numpy as jnp
from jax import lax
from jax.experimental import pallas as pl
from jax.experimental.pallas import tpu as pltpu
from jax.experimental.pallas import tpu_sc as plsc

NUM_CORES = 2
NUM_SUBCORES = 16
NUM_WORKERS = NUM_CORES * NUM_SUBCORES
LANES = 16
CHUNK = 512


def _make_deg_kernel(np_, e, epw):
    rows_per_tile = np_ // NUM_SUBCORES
    vregs_per_tile = rows_per_tile // LANES
    fc = epw // CHUNK
    tail = epw - fc * CHUNK

    mesh = plsc.VectorSubcoreMesh(
        core_axis_name="c", subcore_axis_name="s",
        num_cores=NUM_CORES, num_subcores=NUM_SUBCORES)

    def body(ei_hbm, cnt_hbm, deg_sh, idx_v, ones_v, deg_v, cnt_v, tidx_v,
             isem, ssem):
        c = lax.axis_index("c")
        s = lax.axis_index("s")
        w = s * NUM_CORES + c
        ebase = w * epw

        def zb(i, carry):
            deg_v[pl.ds(i * LANES, LANES)] = jnp.zeros((LANES,), jnp.float32)
            return carry
        lax.fori_loop(0, vregs_per_tile, zb, None)
        pltpu.sync_copy(deg_v, deg_sh.at[pl.ds(s * rows_per_tile, rows_per_tile)])

        def ob(i, carry):
            ones_v[pl.ds(i * LANES, LANES)] = jnp.ones((LANES,), jnp.float32)
            return carry
        lax.fori_loop(0, CHUNK // LANES, ob, None)
        pltpu.sync_copy(ei_hbm.at[1].at[pl.ds(ebase, epw)], idx_v)

        plsc.subcore_barrier()

        def fire(j, carry):
            pltpu.async_copy(ones_v, deg_sh.at[idx_v.at[pl.ds(j * CHUNK, CHUNK)]],
                             ssem, add=True)
            return carry
        lax.fori_loop(0, fc, fire, None)

        def drain(j, carry):
            pltpu.make_async_copy(ones_v, deg_sh.at[idx_v.at[pl.ds(0, CHUNK)]],
                                  ssem).wait()
            return carry
        lax.fori_loop(0, fc, drain, None)

        if tail:
            pltpu.sync_copy(ei_hbm.at[1].at[pl.ds(ebase + fc * CHUNK, tail)],
                            tidx_v)
            pltpu.sync_copy(ones_v.at[pl.ds(0, tail)], deg_sh.at[tidx_v],
                            add=True)

        plsc.subcore_barrier()

        base = s * rows_per_tile
        pltpu.sync_copy(deg_sh.at[pl.ds(base, rows_per_tile)], deg_v)

        def bc(i, carry):
            v = deg_v[pl.ds(i * LANES, LANES)]
            for l in range(LANES):
                cnt_v[i * LANES + l, :] = jnp.full((LANES,), v[l], jnp.float32)
            return carry
        lax.fori_loop(0, vregs_per_tile, bc, None)
        pltpu.sync_copy(cnt_v, cnt_hbm.at[c].at[pl.ds(base, rows_per_tile)])

    return pl.kernel(
        body,
        out_type=jax.ShapeDtypeStruct((NUM_CORES, np_, LANES), jnp.float32),
        mesh=mesh,
        compiler_params=pltpu.CompilerParams(use_tc_tiling_on_sc=False),
        scratch_types=[
            pltpu.VMEM_SHARED((np_,), jnp.float32),
            pltpu.VMEM((epw,), jnp.int32),
            pltpu.VMEM((CHUNK,), jnp.float32),
            pltpu.VMEM((rows_per_tile,), jnp.float32),
            pltpu.VMEM((rows_per_tile, LANES), jnp.float32),
            pltpu.VMEM((tail if tail else LANES,), jnp.int32),
            pltpu.SemaphoreType.DMA,
            pltpu.SemaphoreType.DMA,
        ],
    )


def _make_agg_kernel(np_, e, epw):
    rows_per_tile = np_ // NUM_SUBCORES
    fc = epw // CHUNK
    tail = epw - fc * CHUNK

    mesh = plsc.VectorSubcoreMesh(
        core_axis_name="c", subcore_axis_name="s",
        num_cores=NUM_CORES, num_subcores=NUM_SUBCORES)

    def body(g_hbm, ei_hbm, out_hbm, acc_sh, srcb, dstb,
             rows0, rows1, rows2, rows3, zt, tidx_v, trows, gsem, ssem):
        c = lax.axis_index("c")
        s = lax.axis_index("s")
        w = s * NUM_CORES + c
        ebase = w * epw

        def zb(i, carry):
            zt[i, :] = jnp.zeros((LANES,), jnp.float32)
            return carry
        lax.fori_loop(0, rows_per_tile, zb, None)
        pltpu.sync_copy(zt, acc_sh.at[pl.ds(s * rows_per_tile, rows_per_tile)])
        pltpu.sync_copy(ei_hbm.at[0].at[pl.ds(ebase, epw)], srcb)
        pltpu.sync_copy(ei_hbm.at[1].at[pl.ds(ebase, epw)], dstb)
        plsc.subcore_barrier()

        rows = (rows0, rows1, rows2, rows3)
        nbuf = len(rows)
        depth = nbuf - 1

        def sidx(j):
            return srcb.at[pl.ds(j * CHUNK, CHUNK)]

        def didx(j):
            return dstb.at[pl.ds(j * CHUNK, CHUNK)]

        for k in range(depth):
            pltpu.async_copy(g_hbm.at[sidx(k)], rows[k], gsem)

        def group(jg, carry):
            for b in range(nbuf):
                j = jg * nbuf + b
                buf = rows[b]
                pltpu.make_async_copy(g_hbm.at[sidx(j)], buf, gsem).wait()
                pltpu.async_copy(buf, acc_sh.at[didx(j)], ssem, add=True)
                prv = rows[(b - 1) % nbuf]

                @pl.when(j >= 1)
                def _():
                    pltpu.make_async_copy(prv, acc_sh.at[didx(j - 1)],
                                          ssem).wait()

                @pl.when(j + depth < fc)
                def _():
                    pltpu.async_copy(g_hbm.at[sidx(j + depth)], prv, gsem)
            return carry
        lax.fori_loop(0, fc // nbuf, group, None)
        for j in range(fc - fc % nbuf, fc):
            b = j % nbuf
            pltpu.make_async_copy(g_hbm.at[sidx(j)], rows[b], gsem).wait()
            pltpu.async_copy(rows[b], acc_sh.at[didx(j)], ssem, add=True)
            pltpu.make_async_copy(rows[(b - 1) % nbuf],
                                  acc_sh.at[didx(j - 1)], ssem).wait()
        pltpu.make_async_copy(rows[(fc - 1) % nbuf],
                              acc_sh.at[didx(fc - 1)], ssem).wait()

        if tail:
            pltpu.sync_copy(ei_hbm.at[0].at[pl.ds(ebase + fc * CHUNK, tail)],
                            tidx_v)
            pltpu.async_copy(g_hbm.at[tidx_v], trows, gsem).wait()
            pltpu.sync_copy(ei_hbm.at[1].at[pl.ds(ebase + fc * CHUNK, tail)],
                            tidx_v)
            pltpu.sync_copy(trows, acc_sh.at[tidx_v], add=True)

        plsc.subcore_barrier()
        base = s * rows_per_tile
        pltpu.sync_copy(acc_sh.at[pl.ds(base, rows_per_tile)],
                        out_hbm.at[c].at[pl.ds(base, rows_per_tile)])

    return pl.kernel(
        body,
        out_type=jax.ShapeDtypeStruct((NUM_CORES, np_, LANES), jnp.float32),
        mesh=mesh,
        compiler_params=pltpu.CompilerParams(use_tc_tiling_on_sc=False),
        scratch_types=[
            pltpu.VMEM_SHARED((np_, LANES), jnp.float32),
            pltpu.VMEM((epw,), jnp.int32),
            pltpu.VMEM((epw,), jnp.int32),
            pltpu.VMEM((CHUNK, LANES), jnp.float32),
            pltpu.VMEM((CHUNK, LANES), jnp.float32),
            pltpu.VMEM((CHUNK, LANES), jnp.float32),
            pltpu.VMEM((CHUNK, LANES), jnp.float32),
            pltpu.VMEM((rows_per_tile, LANES), jnp.float32),
            pltpu.VMEM((tail if tail else LANES,), jnp.int32),
            pltpu.VMEM((tail if tail else LANES, LANES), jnp.float32),
            pltpu.SemaphoreType.DMA,
            pltpu.SemaphoreType.DMA,
        ],
    )


_FW = 128
_GRP = _FW // LANES


def _tc_scale_matmul(x8, w1b, cnt_f, nf, npf):
    def body(x_ref, w_ref, c_ref, o_ref, d_ref):
        dis = jax.lax.rsqrt(c_ref[0, :nf] + c_ref[1, :nf] + 1.0)
        d_ref[...] = dis
        h = jnp.dot(x_ref[...], w_ref[...], preferred_element_type=jnp.float32)
        o_ref[...] = dis * h

    return pl.pallas_call(
        body,
        out_shape=[jax.ShapeDtypeStruct((nf, _FW), jnp.float32),
                   jax.ShapeDtypeStruct((nf, _FW), jnp.float32)],
    )(x8, w1b, cnt_f)


def _tc_mid_layer(p_f, g1_f, dis_f, b1, w2b, nf):
    def body(p_ref, g_ref, d_ref, b_ref, w_ref, o_ref):
        agg = p_ref[0, :nf] + p_ref[1, :nf] + g_ref[...]
        h = jnp.maximum(d_ref[...] * agg + b_ref[...], 0.0)
        o_ref[...] = d_ref[...] * jnp.dot(
            h, w_ref[...], preferred_element_type=jnp.float32)

    return pl.pallas_call(
        body,
        out_shape=jax.ShapeDtypeStruct((nf, _FW), jnp.float32),
    )(p_f, g1_f, dis_f, b1, w2b)


def _tc_final_layer(q_f, g2_f, dis_f, b2, nf):
    def body(q_ref, g_ref, d_ref, b_ref, o_ref):
        agg = q_ref[0, :nf] + q_ref[1, :nf] + g_ref[...]
        o_ref[...] = jax.nn.sigmoid(d_ref[...] * agg + b_ref[...])

    return pl.pallas_call(
        body,
        out_shape=jax.ShapeDtypeStruct((nf, _FW), jnp.float32),
    )(q_f, g2_f, dis_f, b2)


def kernel(x, edge_index, W1, b1, W2, b2):
    n, f0 = x.shape
    e = edge_index.shape[1]
    f1 = W1.shape[1]
    f2 = W2.shape[1]

    np_ = -(-(n + LANES) // 256) * 256
    if e % (NUM_WORKERS * LANES):
        e_pad = -(-e // (NUM_WORKERS * LANES)) * NUM_WORKERS * LANES
        padi = jnp.arange(e_pad - e, dtype=edge_index.dtype)
        ei = jnp.concatenate(
            [edge_index, jnp.stack([padi % LANES, n + (padi % LANES)])], axis=1)
    else:
        e_pad = e
        ei = edge_index
    epw = e_pad // NUM_WORKERS
    nf = n * LANES // _FW
    npf = np_ * LANES // _FW

    b1row = jnp.tile(b1, _GRP).reshape(1, _FW)
    w1b = jnp.kron(jnp.eye(_GRP, dtype=jnp.float32), W1)
    w2pad = jnp.zeros((f1, LANES), jnp.float32).at[:, :f2].set(W2)
    w2b = jnp.kron(jnp.eye(_GRP, dtype=jnp.float32), w2pad)
    b2row = jnp.tile(jnp.zeros((LANES,), jnp.float32).at[:f2].set(b2),
                     _GRP).reshape(1, _FW)

    cnt = _make_deg_kernel(np_, e_pad, epw)(ei)
    agg = _make_agg_kernel(np_, e_pad, epw)

    g1_f, dis_f = _tc_scale_matmul(x.reshape(nf, f0 * _GRP), w1b,
                                   cnt.reshape(2, npf, _FW), nf, npf)
    p = agg(g1_f.reshape(n, LANES), ei)
    g2_f = _tc_mid_layer(p.reshape(2, npf, _FW), g1_f, dis_f, b1row, w2b, nf)
    q = agg(g2_f.reshape(n, LANES), ei)
    out_f = _tc_final_layer(q.reshape(2, npf, _FW), g2_f, dis_f, b2row, nf)
    return out_f.reshape(n, LANES)[:, :f2]
